# Initial kernel scaffold; baseline (speedup 1.0000x reference)
import jax
import jax.numpy as jnp
from jax.experimental import pallas as pl


def kernel(feat, edge_index, W0, b0, g0, be0, W1, b1, g1, be1, W2, b2):
    raise NotImplementedError("write your pallas kernel here")


# speedup vs baseline: 2.9465x; 2.9465x over previous
import jax
import jax.numpy as jnp
from jax import lax
from jax.experimental import pallas as pl
from jax.experimental.pallas import tpu as pltpu
from jax.experimental.pallas import tpu_sc as plsc

N = 10000
E = 320000
NC = 2
NS = 16
NW = NC * NS
K = 80
NPAD = 10112
RPS = NPAD // NS

_SC_MESH = dict(core_axis_name="c", subcore_axis_name="s")


def _sc_spmm(dh):
    ept = E // NS
    chunks = ept // K

    def body(x_hbm, src_hbm, dst_hbm, zeros_hbm, out_hbm,
             acc, idx_s, idx_d, rows, blk, sem):
        cid = lax.axis_index("c")
        sid = lax.axis_index("s")
        pltpu.sync_copy(zeros_hbm, blk)
        pltpu.sync_copy(blk, acc.at[pl.ds(sid * RPS, RPS)])
        plsc.subcore_barrier()

        def step(c, carry):
            base = pl.multiple_of(sid * ept + c * K, 16)
            pltpu.sync_copy(src_hbm.at[pl.ds(base, K)], idx_s)
            pltpu.sync_copy(dst_hbm.at[pl.ds(base, K)], idx_d)
            pltpu.async_copy(x_hbm.at[cid].at[idx_s], rows, sem).wait()
            pltpu.sync_copy(rows, acc.at[idx_d], add=True)
            return carry

        lax.fori_loop(0, chunks, step, 0)
        plsc.subcore_barrier()
        pltpu.sync_copy(acc.at[pl.ds(sid * RPS, RPS)], blk)
        pltpu.sync_copy(blk, out_hbm.at[cid, pl.ds(sid * RPS, RPS)])

    return pl.kernel(
        body,
        out_type=jax.ShapeDtypeStruct((NC, NPAD, dh), jnp.float32),
        mesh=plsc.VectorSubcoreMesh(**_SC_MESH),
        compiler_params=pltpu.CompilerParams(use_tc_tiling_on_sc=False),
        scratch_types=[
            pltpu.VMEM_SHARED((NPAD, dh), jnp.float32),
            pltpu.VMEM((K,), jnp.int32),
            pltpu.VMEM((K,), jnp.int32),
            pltpu.VMEM((K, dh), jnp.float32),
            pltpu.VMEM((RPS, dh), jnp.float32),
            pltpu.SemaphoreType.DMA,
        ],
    )


def _sc_degrees():
    epw = E // NW
    chunks = epw // K

    def body(src_hbm, dst_hbm, ones_hbm, zeros_hbm, out_hbm,
             acc, idx_s, idx_d, ones_s, ones_d, blk):
        cid = lax.axis_index("c")
        sid = lax.axis_index("s")
        wid = cid * NS + sid
        pltpu.sync_copy(ones_hbm.at[0], ones_s)
        pltpu.sync_copy(ones_hbm.at[1], ones_d)
        pltpu.sync_copy(zeros_hbm, blk)
        pltpu.sync_copy(blk, acc.at[pl.ds(sid * RPS, RPS)])
        plsc.subcore_barrier()

        def step(c, carry):
            base = pl.multiple_of(wid * epw + c * K, 16)
            pltpu.sync_copy(src_hbm.at[pl.ds(base, K)], idx_s)
            pltpu.sync_copy(dst_hbm.at[pl.ds(base, K)], idx_d)
            pltpu.sync_copy(ones_s, acc.at[idx_s], add=True)
            pltpu.sync_copy(ones_d, acc.at[idx_d], add=True)
            return carry

        lax.fori_loop(0, chunks, step, 0)
        plsc.subcore_barrier()
        pltpu.sync_copy(acc.at[pl.ds(sid * RPS, RPS)], blk)
        pltpu.sync_copy(blk, out_hbm.at[cid, pl.ds(sid * RPS, RPS)])

    return pl.kernel(
        body,
        out_type=jax.ShapeDtypeStruct((NC, NPAD, 8), jnp.float32),
        mesh=plsc.VectorSubcoreMesh(**_SC_MESH),
        compiler_params=pltpu.CompilerParams(use_tc_tiling_on_sc=False),
        scratch_types=[
            pltpu.VMEM_SHARED((NPAD, 8), jnp.float32),
            pltpu.VMEM((K,), jnp.int32),
            pltpu.VMEM((K,), jnp.int32),
            pltpu.VMEM((K, 8), jnp.float32),
            pltpu.VMEM((K, 8), jnp.float32),
            pltpu.VMEM((RPS, 8), jnp.float32),
        ],
    )


def _norms(degp_ref):
    deg = degp_ref[0] + degp_ref[1]
    out_deg = deg[:N, 0:1]
    in_deg = deg[:N, 1:2]
    n_src = lax.rsqrt(jnp.where(out_deg > 0, out_deg, 1.0))
    n_dst = lax.rsqrt(jnp.where(in_deg > 0, in_deg, 1.0))
    return n_src, n_dst


def _split(v, out_ref):
    dh = v.shape[1] // NC
    for c in range(NC):
        out_ref[c] = v[:, c * dh:(c + 1) * dh]


def _tc0(degp_ref, feat_ref, x0_ref):
    n_src, _ = _norms(degp_ref)
    _split(feat_ref[...] * n_src, x0_ref)


def _mid_layer(degp_ref, p_ref, w_ref, b_ref, g_ref, be_ref, out_ref, *,
               w2_ref=None):
    n_src, n_dst = _norms(degp_ref)
    agg = jnp.concatenate([p_ref[0, :N, :], p_ref[1, :N, :]], axis=1)
    t = agg * n_dst
    u = jnp.dot(t, w_ref[...], preferred_element_type=jnp.float32,
                precision=lax.Precision.HIGHEST) + b_ref[...]
    m = jnp.mean(u, axis=0, keepdims=True)
    c = u - m
    var = jnp.mean(c * c, axis=0, keepdims=True)
    v = c * lax.rsqrt(var + 1e-5) * g_ref[...] + be_ref[...]
    v = jnp.maximum(v, 0.0) * n_src
    if w2_ref is not None:
        v = jnp.dot(v, w2_ref[...], preferred_element_type=jnp.float32,
                    precision=lax.Precision.HIGHEST)
        pad = out_ref.shape[0] * out_ref.shape[2] - v.shape[1]
        if pad:
            v = jnp.pad(v, ((0, 0), (0, pad)))
    _split(v, out_ref)


def _tc1(degp_ref, p_ref, w_ref, b_ref, g_ref, be_ref, out_ref):
    _mid_layer(degp_ref, p_ref, w_ref, b_ref, g_ref, be_ref, out_ref)


def _tc2(degp_ref, p_ref, w_ref, b_ref, g_ref, be_ref, w2_ref, out_ref):
    _mid_layer(degp_ref, p_ref, w_ref, b_ref, g_ref, be_ref, out_ref,
               w2_ref=w2_ref)


def _tc3(degp_ref, q_ref, b2_ref, out_ref):
    _, n_dst = _norms(degp_ref)
    agg = jnp.concatenate([q_ref[0, :N, :], q_ref[1, :N, :]], axis=1)
    out_ref[...] = agg[:, :out_ref.shape[1]] * n_dst + b2_ref[...]


def kernel(feat, edge_index, W0, b0, g0, be0, W1, b1, g1, be1, W2, b2):
    src = edge_index[0]
    dst = edge_index[1]
    d_hid = W0.shape[1]
    n_cls = W2.shape[1]
    dh_h = d_hid // NC
    dh_c = (-(-n_cls // (8 * NC))) * 8

    zeros_h = jnp.zeros((RPS, dh_h), jnp.float32)
    zeros_c = jnp.zeros((RPS, dh_c), jnp.float32)
    zeros_8 = jnp.zeros((RPS, 8), jnp.float32)
    ones_8 = (jnp.zeros((2, K, 8), jnp.float32)
              .at[0, :, 0].set(1.0).at[1, :, 1].set(1.0))

    degp = _sc_degrees()(src, dst, ones_8, zeros_8)

    x0 = pl.pallas_call(
        _tc0, out_shape=jax.ShapeDtypeStruct((NC, N, dh_h), jnp.float32),
    )(degp, feat)

    spmm_h = _sc_spmm(dh_h)
    spmm_c = _sc_spmm(dh_c)

    p0 = spmm_h(x0, src, dst, zeros_h)
    x1 = pl.pallas_call(
        _tc1, out_shape=jax.ShapeDtypeStruct((NC, N, dh_h), jnp.float32),
    )(degp, p0, W0, b0, g0, be0)

    p1 = spmm_h(x1, src, dst, zeros_h)
    y2 = pl.pallas_call(
        _tc2, out_shape=jax.ShapeDtypeStruct((NC, N, dh_c), jnp.float32),
    )(degp, p1, W1, b1, g1, be1, W2)

    q = spmm_c(y2, src, dst, zeros_c)
    out = pl.pallas_call(
        _tc3, out_shape=jax.ShapeDtypeStruct((N, n_cls), jnp.float32),
    )(degp, q, b2)
    return out


# speedup vs baseline: 11.6215x; 3.9441x over previous
import jax
import jax.numpy as jnp
from jax import lax
from jax.experimental import pallas as pl
from jax.experimental.pallas import tpu as pltpu
from jax.experimental.pallas import tpu_sc as plsc

N = 10000
E = 320000
NC = 2
NS = 16
NW = NC * NS
K = 80
NPAD = 10112
RPS = NPAD // NS

_SC_MESH = dict(core_axis_name="c", subcore_axis_name="s")


NBUF = 5
SCK = 10


def _sc_spmm(dh):
    ept = E // NS
    cps = ept // K
    nsc = cps // SCK

    def body(x_hbm, src_hbm, dst_hbm, zeros_hbm, out_hbm,
             acc, idx_s, idx_d, rows, sems):
        cid = lax.axis_index("c")
        sid = lax.axis_index("s")
        pltpu.sync_copy(zeros_hbm, acc.at[pl.ds(sid * RPS, RPS)])
        pltpu.sync_copy(src_hbm.at[sid], idx_s)
        pltpu.sync_copy(dst_hbm.at[sid], idx_d)
        plsc.subcore_barrier()

        def gather(c, b):
            pltpu.async_copy(x_hbm.at[cid].at[idx_s.at[c]], rows[b], sems[b])

        def drain_scatter(c, b):
            pltpu.make_async_copy(
                x_hbm.at[cid].at[idx_s.at[c]], rows[b], sems[b]).wait()
            pltpu.sync_copy(rows[b], acc.at[idx_d.at[c]], add=True)

        for j in range(NBUF):
            gather(j, j % NBUF)

        def step(s, carry):
            c0 = s * SCK
            for j in range(SCK):
                drain_scatter(c0 + j, j % NBUF)
                gather(c0 + j + NBUF, j % NBUF)
            return carry

        lax.fori_loop(0, nsc - 1, step, 0)
        c0 = (nsc - 1) * SCK
        for j in range(SCK):
            drain_scatter(c0 + j, j % NBUF)
            if c0 + j + NBUF < cps:
                gather(c0 + j + NBUF, j % NBUF)
        plsc.subcore_barrier()
        pltpu.sync_copy(acc.at[pl.ds(sid * RPS, RPS)],
                        out_hbm.at[cid, pl.ds(sid * RPS, RPS)])

    return pl.kernel(
        body,
        out_type=jax.ShapeDtypeStruct((NC, NPAD, dh), jnp.float32),
        mesh=plsc.VectorSubcoreMesh(**_SC_MESH),
        compiler_params=pltpu.CompilerParams(use_tc_tiling_on_sc=False),
        scratch_types=[
            pltpu.VMEM_SHARED((NPAD, dh), jnp.float32),
            pltpu.VMEM((cps, K), jnp.int32),
            pltpu.VMEM((cps, K), jnp.int32),
            [pltpu.VMEM((K, dh), jnp.float32) for _ in range(NBUF)],
            [pltpu.SemaphoreType.DMA for _ in range(NBUF)],
        ],
    )


def _sc_degrees():
    epw = E // NW
    cps = epw // K
    sck = 5
    nsc = cps // sck

    def body(src_hbm, dst_hbm, ones_hbm, zeros_hbm, out_hbm,
             acc, idx_s, idx_d, ones_s, ones_d, sem):
        cid = lax.axis_index("c")
        sid = lax.axis_index("s")
        wid = cid * NS + sid
        pltpu.sync_copy(ones_hbm.at[0], ones_s)
        pltpu.sync_copy(ones_hbm.at[1], ones_d)
        pltpu.sync_copy(zeros_hbm, acc.at[pl.ds(sid * RPS, RPS)])
        pltpu.sync_copy(src_hbm.at[wid], idx_s)
        pltpu.sync_copy(dst_hbm.at[wid], idx_d)
        plsc.subcore_barrier()

        def issue(c):
            pltpu.async_copy(ones_s, acc.at[idx_s.at[c]], sem, add=True)
            pltpu.async_copy(ones_d, acc.at[idx_d.at[c]], sem, add=True)

        def drain(c):
            pltpu.make_async_copy(ones_s, acc.at[idx_s.at[c]], sem).wait()
            pltpu.make_async_copy(ones_d, acc.at[idx_d.at[c]], sem).wait()

        for j in range(sck):
            issue(j)

        def step(s, carry):
            c0 = s * sck
            for j in range(sck):
                drain(c0 + j)
                issue(c0 + j + sck)
            return carry

        lax.fori_loop(0, nsc - 1, step, 0)
        c0 = (nsc - 1) * sck
        for j in range(sck):
            drain(c0 + j)
        plsc.subcore_barrier()
        pltpu.sync_copy(acc.at[pl.ds(sid * RPS, RPS)],
                        out_hbm.at[cid, pl.ds(sid * RPS, RPS)])

    return pl.kernel(
        body,
        out_type=jax.ShapeDtypeStruct((NC, NPAD, 8), jnp.float32),
        mesh=plsc.VectorSubcoreMesh(**_SC_MESH),
        compiler_params=pltpu.CompilerParams(use_tc_tiling_on_sc=False),
        scratch_types=[
            pltpu.VMEM_SHARED((NPAD, 8), jnp.float32),
            pltpu.VMEM((cps, K), jnp.int32),
            pltpu.VMEM((cps, K), jnp.int32),
            pltpu.VMEM((K, 8), jnp.float32),
            pltpu.VMEM((K, 8), jnp.float32),
            pltpu.SemaphoreType.DMA,
        ],
    )


def _norms(degp_ref):
    deg = degp_ref[0] + degp_ref[1]
    out_deg = deg[:N, 0:1]
    in_deg = deg[:N, 1:2]
    n_src = lax.rsqrt(jnp.where(out_deg > 0, out_deg, 1.0))
    n_dst = lax.rsqrt(jnp.where(in_deg > 0, in_deg, 1.0))
    return n_src, n_dst


def _split(v, out_ref):
    dh = v.shape[1] // NC
    for c in range(NC):
        out_ref[c] = v[:, c * dh:(c + 1) * dh]


def _tc0(degp_ref, feat_ref, x0_ref):
    n_src, _ = _norms(degp_ref)
    _split(feat_ref[...] * n_src, x0_ref)


def _mid_layer(degp_ref, p_ref, w_ref, b_ref, g_ref, be_ref, out_ref, *,
               w2_ref=None):
    n_src, n_dst = _norms(degp_ref)
    agg = jnp.concatenate([p_ref[0, :N, :], p_ref[1, :N, :]], axis=1)
    t = agg * n_dst
    u = jnp.dot(t, w_ref[...], preferred_element_type=jnp.float32,
                precision=lax.Precision.HIGHEST) + b_ref[...]
    m = jnp.mean(u, axis=0, keepdims=True)
    c = u - m
    var = jnp.mean(c * c, axis=0, keepdims=True)
    v = c * lax.rsqrt(var + 1e-5) * g_ref[...] + be_ref[...]
    v = jnp.maximum(v, 0.0) * n_src
    if w2_ref is not None:
        v = jnp.dot(v, w2_ref[...], preferred_element_type=jnp.float32,
                    precision=lax.Precision.HIGHEST)
        pad = out_ref.shape[0] * out_ref.shape[2] - v.shape[1]
        if pad:
            v = jnp.pad(v, ((0, 0), (0, pad)))
    _split(v, out_ref)


def _tc1(degp_ref, p_ref, w_ref, b_ref, g_ref, be_ref, out_ref):
    _mid_layer(degp_ref, p_ref, w_ref, b_ref, g_ref, be_ref, out_ref)


def _tc2(degp_ref, p_ref, w_ref, b_ref, g_ref, be_ref, w2_ref, out_ref):
    _mid_layer(degp_ref, p_ref, w_ref, b_ref, g_ref, be_ref, out_ref,
               w2_ref=w2_ref)


def _tc3(degp_ref, q_ref, b2_ref, out_ref):
    _, n_dst = _norms(degp_ref)
    agg = jnp.concatenate([q_ref[0, :N, :], q_ref[1, :N, :]], axis=1)
    out_ref[...] = agg[:, :out_ref.shape[1]] * n_dst + b2_ref[...]


def kernel(feat, edge_index, W0, b0, g0, be0, W1, b1, g1, be1, W2, b2):
    src = edge_index[0]
    dst = edge_index[1]
    d_hid = W0.shape[1]
    n_cls = W2.shape[1]
    dh_h = d_hid // NC
    dh_c = (-(-n_cls // (8 * NC))) * 8

    zeros_h = jnp.zeros((RPS, dh_h), jnp.float32)
    zeros_c = jnp.zeros((RPS, dh_c), jnp.float32)
    zeros_8 = jnp.zeros((RPS, 8), jnp.float32)
    ones_8 = (jnp.zeros((2, K, 8), jnp.float32)
              .at[0, :, 0].set(1.0).at[1, :, 1].set(1.0))

    src_w = src.reshape(NW, -1, K)
    dst_w = dst.reshape(NW, -1, K)
    src_s = src.reshape(NS, -1, K)
    dst_s = dst.reshape(NS, -1, K)

    degp = _sc_degrees()(src_w, dst_w, ones_8, zeros_8)

    x0 = pl.pallas_call(
        _tc0, out_shape=jax.ShapeDtypeStruct((NC, N, dh_h), jnp.float32),
    )(degp, feat)

    spmm_h = _sc_spmm(dh_h)
    spmm_c = _sc_spmm(dh_c)

    p0 = spmm_h(x0, src_s, dst_s, zeros_h)
    x1 = pl.pallas_call(
        _tc1, out_shape=jax.ShapeDtypeStruct((NC, N, dh_h), jnp.float32),
    )(degp, p0, W0, b0, g0, be0)

    p1 = spmm_h(x1, src_s, dst_s, zeros_h)
    y2 = pl.pallas_call(
        _tc2, out_shape=jax.ShapeDtypeStruct((NC, N, dh_c), jnp.float32),
    )(degp, p1, W1, b1, g1, be1, W2)

    q = spmm_c(y2, src_s, dst_s, zeros_c)
    out = pl.pallas_call(
        _tc3, out_shape=jax.ShapeDtypeStruct((N, n_cls), jnp.float32),
    )(degp, q, b2)
    return out
